# baseline (speedup 1.0000x reference)
import jax
import jax.numpy as jnp
from jax.experimental import pallas as pl
from jax.experimental.pallas import tpu as pltpu

BLK = 256
H1 = 32
W1 = 30
H2 = 16
NL = 128


NSPLIT = 4


def _body(x_hbm, w1_ref, b1_ref, w2_ref, b2_ref, wfc_ref, fcb_ref, o_ref,
          xb_ref, s0_ref, s_ref, a1_ref, s3_ref, f_ref, sem):
    m0 = BLK * H1 - 2
    m1 = BLK * H2 - 2
    r1 = BLK * H2
    r2 = BLK * 8

    step = pl.program_id(0)
    nstep = pl.num_programs(0)
    sz = BLK // NSPLIT

    def _start(st, slot):
        for h in range(NSPLIT):
            pltpu.make_async_copy(
                x_hbm.at[pl.ds(st * BLK + h * sz, sz)],
                xb_ref.at[slot, pl.ds(h * sz, sz)],
                sem.at[slot, h]).start()

    def _wait(slot):
        for h in range(NSPLIT):
            pltpu.make_async_copy(
                x_hbm.at[pl.ds(0, sz)],
                xb_ref.at[slot, pl.ds(0, sz)],
                sem.at[slot, h]).wait()

    cur = jax.lax.rem(step, 2)
    nxt = jax.lax.rem(step + 1, 2)

    @pl.when(step == 0)
    def _():
        _start(0, 0)

    @pl.when(step + 1 < nstep)
    def _():
        _start(step + 1, nxt)

    _wait(cur)
    x_ref = xb_ref.at[cur]

    s0_ref[pl.ds(8, 1), pl.ds(0, 32)] = jnp.zeros((1, 32), jnp.float32)
    zrow = jnp.zeros((4, 32), jnp.float32)
    for b in range(BLK):
        v = x_ref[b, 0]
        v32 = jnp.concatenate(
            [jnp.pad(v, ((0, 0), (0, 4))), zrow], axis=0)
        base = 8 + b * H1
        s0_ref[pl.ds(base + 1, 32), pl.ds(0, 32)] = v32
        s0_ref[pl.ds(base + 0, 32), pl.ds(32, 32)] = v32
        s0_ref[pl.ds(base - 1, 32), pl.ds(64, 64)] = jnp.pad(
            v32, ((0, 0), (0, 32)))

    h1 = jnp.dot(s0_ref[pl.ds(8, m0), :], w1_ref[...],
                 preferred_element_type=jnp.float32)
    s_ref[pl.ds(0, m0), :] = jnp.maximum(h1[:, :NL], h1[:, NL:]) + b1_ref[...]

    ev = s_ref[pl.ds(0, r1, 2), :]
    od = s_ref[pl.ds(1, r1, 2), :]
    row16 = jax.lax.broadcasted_iota(jnp.int32, (r1, NL), 0) & (H2 - 1)
    a1 = jnp.where(row16 < 14, jnp.maximum(jnp.maximum(ev, od), 0.0), 0.0)
    a1_ref[pl.ds(1, r1), :] = a1
    a1_ref[pl.ds(0, 1), :] = jnp.zeros((1, NL), jnp.float32)

    a1c = jnp.concatenate(
        [a1_ref[pl.ds(ky, m1), :] for ky in (0, 1, 2)], axis=1)
    h2 = jnp.dot(a1c, w2_ref[...], preferred_element_type=jnp.float32)
    s3_ref[pl.ds(0, m1), :] = jnp.maximum(h2[:, :NL], h2[:, NL:]) + b2_ref[...]

    ev2 = s3_ref[pl.ds(0, r2, 2), :]
    od2 = s3_ref[pl.ds(1, r2, 2), :]
    row8 = jax.lax.broadcasted_iota(jnp.int32, (r2, NL), 0) & 7
    f_ref[...] = jnp.where(row8 < 7,
                           jnp.maximum(jnp.maximum(ev2, od2), 0.0), 0.0)

    fc = jnp.concatenate(
        [f_ref[pl.ds(y2, BLK, 8), :] for y2 in range(7)], axis=1)
    res = jnp.dot(fc, wfc_ref[...],
                  preferred_element_type=jnp.float32) + fcb_ref[...]
    o_ref[...] = res[:, :10]


def kernel(x, w1, b1, s1, w2, b2, s2, wfc, fcb):
    del s1, s2
    n = x.shape[0]
    n_pad = ((n + BLK - 1) // BLK) * BLK
    xr = x
    if n_pad != n:
        xr = jnp.pad(xr, ((0, n_pad - n), (0, 0), (0, 0), (0, 0)))

    w1p = jnp.concatenate(
        [jnp.pad(w1[t, 1:29, :], ((0, 4), (0, 0))) for t in range(3)]
        + [jnp.zeros((32, 2 * NL), w1.dtype)], axis=0)
    w2s = jnp.concatenate([w2[0], w2[1], w2[2]], axis=0)
    wfcs = jnp.concatenate([wfc[i] for i in range(7)], axis=0)

    logits = pl.pallas_call(
        _body,
        out_shape=jax.ShapeDtypeStruct((n_pad, 10), jnp.float32),
        grid=(n_pad // BLK,),
        in_specs=[
            pl.BlockSpec(memory_space=pltpu.MemorySpace.HBM),
            pl.BlockSpec((NL, 2 * NL), lambda i: (0, 0)),
            pl.BlockSpec((1, NL), lambda i: (0, 0)),
            pl.BlockSpec((3 * NL, 2 * NL), lambda i: (0, 0)),
            pl.BlockSpec((1, NL), lambda i: (0, 0)),
            pl.BlockSpec((7 * NL, NL), lambda i: (0, 0)),
            pl.BlockSpec((1, NL), lambda i: (0, 0)),
        ],
        out_specs=pl.BlockSpec((BLK, 10), lambda i: (i, 0)),
        scratch_shapes=[
            pltpu.VMEM((2, BLK, 1, 28, 28), jnp.float32),
            pltpu.VMEM((BLK * H1 + 16, NL), jnp.float32),
            pltpu.VMEM((BLK * H1, NL), jnp.float32),
            pltpu.VMEM((BLK * H2 + 8, NL), jnp.float32),
            pltpu.VMEM((BLK * H2, NL), jnp.float32),
            pltpu.VMEM((BLK * 8, NL), jnp.float32),
            pltpu.SemaphoreType.DMA((2, NSPLIT)),
        ],
        compiler_params=pltpu.CompilerParams(
            dimension_semantics=("parallel",)),
    )(xr, w1p, b1, w2s, b2, wfcs, fcb)
    return logits[:n]


# speedup vs baseline: 1.0252x; 1.0252x over previous
import jax
import jax.numpy as jnp
from jax.experimental import pallas as pl
from jax.experimental.pallas import tpu as pltpu

BLK = 512
H1 = 32
W1 = 30
H2 = 16
NL = 128


NSPLIT = 4


def _body(x_hbm, w1_ref, b1_ref, w2_ref, b2_ref, wfc_ref, fcb_ref, o_ref,
          xb_ref, s0_ref, s_ref, a1_ref, s3_ref, f_ref, sem):
    m0 = BLK * H1 - 2
    m1 = BLK * H2 - 2
    r1 = BLK * H2
    r2 = BLK * 8

    step = pl.program_id(0)
    nstep = pl.num_programs(0)
    sz = BLK // NSPLIT

    def _start(st, slot):
        for h in range(NSPLIT):
            pltpu.make_async_copy(
                x_hbm.at[pl.ds(st * BLK + h * sz, sz)],
                xb_ref.at[slot, pl.ds(h * sz, sz)],
                sem.at[slot, h]).start()

    def _wait(slot):
        for h in range(NSPLIT):
            pltpu.make_async_copy(
                x_hbm.at[pl.ds(0, sz)],
                xb_ref.at[slot, pl.ds(0, sz)],
                sem.at[slot, h]).wait()

    cur = jax.lax.rem(step, 2)
    nxt = jax.lax.rem(step + 1, 2)

    @pl.when(step == 0)
    def _():
        _start(0, 0)

    @pl.when(step + 1 < nstep)
    def _():
        _start(step + 1, nxt)

    _wait(cur)
    x_ref = xb_ref.at[cur]

    s0_ref[pl.ds(8, 1), pl.ds(0, 32)] = jnp.zeros((1, 32), jnp.float32)
    for b in range(BLK):
        v = x_ref[b, 0]
        v32 = jnp.pad(v, ((0, 4), (0, 4)))
        base = 8 + b * H1
        s0_ref[pl.ds(base + 1, 32), pl.ds(0, 32)] = v32
        s0_ref[pl.ds(base + 0, 32), pl.ds(32, 32)] = v32
        s0_ref[pl.ds(base - 1, 32), pl.ds(64, 64)] = jnp.pad(
            v32, ((0, 0), (0, 32)))

    h1 = jnp.dot(s0_ref[pl.ds(8, m0), :], w1_ref[...],
                 preferred_element_type=jnp.float32)
    s_ref[pl.ds(0, m0), :] = jnp.maximum(h1[:, :NL], h1[:, NL:]) + b1_ref[...]

    ev = s_ref[pl.ds(0, r1, 2), :]
    od = s_ref[pl.ds(1, r1, 2), :]
    row16 = jax.lax.broadcasted_iota(jnp.int32, (r1, NL), 0) & (H2 - 1)
    a1 = jnp.where(row16 < 14, jnp.maximum(jnp.maximum(ev, od), 0.0), 0.0)
    a1_ref[pl.ds(1, r1), :] = a1
    a1_ref[pl.ds(0, 1), :] = jnp.zeros((1, NL), jnp.float32)

    a1c = jnp.concatenate(
        [a1_ref[pl.ds(ky, m1), :] for ky in (0, 1, 2)], axis=1)
    h2 = jnp.dot(a1c, w2_ref[...], preferred_element_type=jnp.float32)
    s3_ref[pl.ds(0, m1), :] = jnp.maximum(h2[:, :NL], h2[:, NL:]) + b2_ref[...]

    ev2 = s3_ref[pl.ds(0, r2, 2), :]
    od2 = s3_ref[pl.ds(1, r2, 2), :]
    row8 = jax.lax.broadcasted_iota(jnp.int32, (r2, NL), 0) & 7
    f_ref[...] = jnp.where(row8 < 7,
                           jnp.maximum(jnp.maximum(ev2, od2), 0.0), 0.0)

    fc = jnp.concatenate(
        [f_ref[pl.ds(y2, BLK, 8), :] for y2 in range(7)], axis=1)
    res = jnp.dot(fc, wfc_ref[...],
                  preferred_element_type=jnp.float32) + fcb_ref[...]
    o_ref[...] = res[:, :10]


def kernel(x, w1, b1, s1, w2, b2, s2, wfc, fcb):
    del s1, s2
    n = x.shape[0]
    n_pad = ((n + BLK - 1) // BLK) * BLK
    xr = x
    if n_pad != n:
        xr = jnp.pad(xr, ((0, n_pad - n), (0, 0), (0, 0), (0, 0)))

    w1p = jnp.concatenate(
        [jnp.pad(w1[t, 1:29, :], ((0, 4), (0, 0))) for t in range(3)]
        + [jnp.zeros((32, 2 * NL), w1.dtype)], axis=0)
    w2s = jnp.concatenate([w2[0], w2[1], w2[2]], axis=0)
    wfcs = jnp.concatenate([wfc[i] for i in range(7)], axis=0)

    logits = pl.pallas_call(
        _body,
        out_shape=jax.ShapeDtypeStruct((n_pad, 10), jnp.float32),
        grid=(n_pad // BLK,),
        in_specs=[
            pl.BlockSpec(memory_space=pltpu.MemorySpace.HBM),
            pl.BlockSpec((NL, 2 * NL), lambda i: (0, 0)),
            pl.BlockSpec((1, NL), lambda i: (0, 0)),
            pl.BlockSpec((3 * NL, 2 * NL), lambda i: (0, 0)),
            pl.BlockSpec((1, NL), lambda i: (0, 0)),
            pl.BlockSpec((7 * NL, NL), lambda i: (0, 0)),
            pl.BlockSpec((1, NL), lambda i: (0, 0)),
        ],
        out_specs=pl.BlockSpec((BLK, 10), lambda i: (i, 0)),
        scratch_shapes=[
            pltpu.VMEM((2, BLK, 1, 28, 28), jnp.float32),
            pltpu.VMEM((BLK * H1 + 16, NL), jnp.float32),
            pltpu.VMEM((BLK * H1, NL), jnp.float32),
            pltpu.VMEM((BLK * H2 + 8, NL), jnp.float32),
            pltpu.VMEM((BLK * H2, NL), jnp.float32),
            pltpu.VMEM((BLK * 8, NL), jnp.float32),
            pltpu.SemaphoreType.DMA((2, NSPLIT)),
        ],
        compiler_params=pltpu.CompilerParams(
            dimension_semantics=("parallel",)),
    )(xr, w1p, b1, w2s, b2, wfcs, fcb)
    return logits[:n]


# speedup vs baseline: 1.0270x; 1.0018x over previous
import jax
import jax.numpy as jnp
from jax.experimental import pallas as pl
from jax.experimental.pallas import tpu as pltpu

BLK = 512
H1 = 32
W1 = 30
H2 = 16
NL = 128


NSPLIT = 4


def _body(x_hbm, w1_ref, b1_ref, w2_ref, b2_ref, wfc_ref, fcb_ref, o_ref,
          xb_ref, s0_ref, s_ref, a1_ref, s3_ref, f_ref, sem):
    m0 = BLK * H1 - 2
    m1 = BLK * H2 - 2
    r1 = BLK * H2
    r2 = BLK * 8

    step = pl.program_id(0)
    nstep = pl.num_programs(0)
    sz = BLK // NSPLIT

    def _start(st, slot):
        for h in range(NSPLIT):
            pltpu.make_async_copy(
                x_hbm.at[pl.ds(st * BLK + h * sz, sz)],
                xb_ref.at[slot, pl.ds(h * sz, sz)],
                sem.at[slot, h]).start()

    def _wait(slot):
        for h in range(NSPLIT):
            pltpu.make_async_copy(
                x_hbm.at[pl.ds(0, sz)],
                xb_ref.at[slot, pl.ds(0, sz)],
                sem.at[slot, h]).wait()

    cur = jax.lax.rem(step, 2)
    nxt = jax.lax.rem(step + 1, 2)

    @pl.when(step == 0)
    def _():
        _start(0, 0)

    @pl.when(step + 1 < nstep)
    def _():
        _start(step + 1, nxt)

    _wait(cur)
    x_ref = xb_ref.at[cur]

    s0_ref[pl.ds(8, 1), pl.ds(0, 32)] = jnp.zeros((1, 32), jnp.float32)
    for b in range(BLK):
        v = x_ref[b, 0]
        v32 = jnp.pad(v, ((0, 4), (0, 4)))
        base = 8 + b * H1
        s0_ref[pl.ds(base + 1, 32), pl.ds(0, 32)] = v32
        s0_ref[pl.ds(base + 0, 32), pl.ds(32, 32)] = v32
        s0_ref[pl.ds(base - 1, 32), pl.ds(64, 64)] = jnp.pad(
            v32, ((0, 0), (0, 32)))

    h1 = jnp.dot(s0_ref[pl.ds(8, m0), :], w1_ref[...],
                 preferred_element_type=jnp.float32)
    s_ref[pl.ds(0, m0), :] = jnp.maximum(h1[:, :NL], h1[:, NL:]) + b1_ref[...]

    ev = s_ref[pl.ds(0, r1, 2), :]
    od = s_ref[pl.ds(1, r1, 2), :]
    row16 = jax.lax.broadcasted_iota(jnp.int32, (r1, NL), 0) & (H2 - 1)
    a1 = jnp.where(row16 < 14, jnp.maximum(jnp.maximum(ev, od), 0.0), 0.0)
    a1_ref[pl.ds(1, r1), :] = a1
    a1_ref[pl.ds(0, 1), :] = jnp.zeros((1, NL), jnp.float32)

    a1c = jnp.concatenate(
        [a1_ref[pl.ds(ky, m1), :] for ky in (0, 1, 2)], axis=1)
    h2 = jnp.dot(a1c, w2_ref[...], preferred_element_type=jnp.float32)
    s3_ref[pl.ds(0, m1), :] = jnp.maximum(h2[:, :NL], h2[:, NL:]) + b2_ref[...]

    ev2 = s3_ref[pl.ds(0, r2, 2), :]
    od2 = s3_ref[pl.ds(1, r2, 2), :]
    row8 = jax.lax.broadcasted_iota(jnp.int32, (r2, NL), 0) & 7
    f_ref[...] = jnp.where(row8 < 7,
                           jnp.maximum(jnp.maximum(ev2, od2), 0.0), 0.0)

    fc = jnp.concatenate(
        [f_ref[pl.ds(y2, BLK, 8), :] for y2 in range(7)], axis=1)
    res = jnp.dot(fc, wfc_ref[...],
                  preferred_element_type=jnp.float32) + fcb_ref[...]
    o_ref[...] = res[:, :10]


def kernel(x, w1, b1, s1, w2, b2, s2, wfc, fcb):
    del s1, s2
    n = x.shape[0]
    n_pad = ((n + BLK - 1) // BLK) * BLK
    xr = x
    if n_pad != n:
        xr = jnp.pad(xr, ((0, n_pad - n), (0, 0), (0, 0), (0, 0)))

    w1p = jnp.concatenate(
        [jnp.pad(w1[t, 1:29, :], ((0, 4), (0, 0))) for t in range(3)]
        + [jnp.zeros((32, 2 * NL), w1.dtype)], axis=0)
    w2s = jnp.concatenate([w2[0], w2[1], w2[2]], axis=0)
    wfcs = jnp.concatenate([wfc[i] for i in range(7)], axis=0)

    logits = pl.pallas_call(
        _body,
        out_shape=jax.ShapeDtypeStruct((n_pad, 10), jnp.float32),
        grid=(n_pad // BLK,),
        in_specs=[
            pl.BlockSpec(memory_space=pltpu.MemorySpace.HBM),
            pl.BlockSpec((NL, 2 * NL), lambda i: (0, 0)),
            pl.BlockSpec((1, NL), lambda i: (0, 0)),
            pl.BlockSpec((3 * NL, 2 * NL), lambda i: (0, 0)),
            pl.BlockSpec((1, NL), lambda i: (0, 0)),
            pl.BlockSpec((7 * NL, NL), lambda i: (0, 0)),
            pl.BlockSpec((1, NL), lambda i: (0, 0)),
        ],
        out_specs=pl.BlockSpec((BLK, 10), lambda i: (i, 0)),
        scratch_shapes=[
            pltpu.VMEM((2, BLK, 1, 28, 28), jnp.float32),
            pltpu.VMEM((BLK * H1 + 16, NL), jnp.float32),
            pltpu.VMEM((BLK * H1, NL), jnp.float32),
            pltpu.VMEM((BLK * H2 + 8, NL), jnp.float32),
            pltpu.VMEM((BLK * H2, NL), jnp.float32),
            pltpu.VMEM((BLK * 8, NL), jnp.float32),
            pltpu.SemaphoreType.DMA((2, NSPLIT)),
        ],
        compiler_params=pltpu.CompilerParams(
            dimension_semantics=("arbitrary",)),
    )(xr, w1p, b1, w2s, b2, wfcs, fcb)
    return logits[:n]
